# baseline (speedup 1.0000x reference)
import jax
import jax.numpy as jnp
from jax import lax
from jax.experimental import pallas as pl
from jax.experimental.pallas import tpu as pltpu
from jax.experimental.pallas import tpu_sc as plsc

N = 4096
NBUCKETS = 1024
NTILES = 16
PER_TILE = N // NTILES
STRIPE = NBUCKETS // NTILES
LN2 = 0.6931471805599453
EPSILON = 1e-07


def _lane(v, i):
    dn = lax.GatherDimensionNumbers(offset_dims=(), collapsed_slice_dims=(0,),
                                    start_index_map=(0,))
    idx = jnp.full((16, 1), i, jnp.int32)
    return lax.gather(v, idx, dn, slice_sizes=(1,),
                      mode=lax.GatherScatterMode.PROMISE_IN_BOUNDS)


def _log16(s):
    bits = plsc.bitcast(s, jnp.int32)
    ex = ((bits >> 23) & 0xFF) - 127
    m = plsc.bitcast((bits & 0x7FFFFF) | 0x3F800000, jnp.float32)
    big = m > 1.5
    m = jnp.where(big, m * 0.5, m)
    ex = jnp.where(big, ex + 1, ex)
    z = (m - 1.0) / (m + 1.0)
    z2 = z * z
    p = jnp.full((16,), 1.0 / 9.0, jnp.float32)
    for c in (1.0 / 7.0, 1.0 / 5.0, 1.0 / 3.0, 1.0):
        p = p * z2 + c
    return ex.astype(jnp.float32) * LN2 + 2.0 * z * p


def _cox_body(h_hbm, ie_hbm, et_hbm, out_hbm,
              h_v, ie_v, idx_a, idx_b, e_a, e_b, idx16_v,
              z_v, loc_v, sfx_v, tot_v, offs_v, stage_v, out_v,
              hist_s, sfx_s, tot_s, pacc_s,
              sem_a, sem_b, sem_c, sem_d):
    wid = lax.axis_index("s")
    base = wid * PER_TILE
    lanes = lax.iota(jnp.int32, 16)
    zeros16 = jnp.zeros((16,), jnp.float32)

    cp_h = pltpu.async_copy(h_hbm.at[pl.ds(base, PER_TILE)], h_v, sem_a)
    cp_i1 = pltpu.async_copy(et_hbm.at[pl.ds(base, 128)], idx_a, sem_b)
    cp_i2 = pltpu.async_copy(et_hbm.at[pl.ds(base + 128, 128)], idx_b, sem_b)
    cp_ie = pltpu.async_copy(ie_hbm.at[pl.ds(base, PER_TILE)], ie_v, sem_c)

    for k in range(4):
        z_v[pl.ds(k * 16, 16)] = zeros16
    idx16_v[...] = lanes
    cp_z = pltpu.async_copy(z_v, hist_s.at[pl.ds(wid * STRIPE, STRIPE)], sem_d)

    @pl.when(wid == 0)
    def _():
        pltpu.sync_copy(z_v.at[pl.ds(0, 16)], pacc_s)

    cp_h.wait()
    for c in range(PER_TILE // 16):
        ec = jnp.exp(h_v[pl.ds(c * 16, 16)])
        if c < 8:
            e_a[pl.ds(c * 16, 16)] = ec
        else:
            e_b[pl.ds((c - 8) * 16, 16)] = ec
    cp_z.wait()

    plsc.subcore_barrier()

    cp_i1.wait()
    cp_i2.wait()
    s1 = pltpu.async_copy(e_a, hist_s.at[idx_a], sem_d, add=True)
    s2 = pltpu.async_copy(e_b, hist_s.at[idx_b], sem_d, add=True)
    s1.wait()
    s2.wait()

    plsc.subcore_barrier()

    pltpu.sync_copy(hist_s.at[pl.ds(wid * STRIPE, STRIPE)], loc_v)
    carry = zeros16
    for c in range(STRIPE // 16 - 1, -1, -1):
        v = loc_v[pl.ds(c * 16, 16)]
        cs = plsc.cumsum(lax.rev(v, (0,)))
        loc_v[pl.ds(c * 16, 16)] = lax.rev(cs, (0,)) + carry
        carry = carry + _lane(cs, 15)
    cp_sf = pltpu.async_copy(loc_v, sfx_s.at[pl.ds(wid * STRIPE, STRIPE)],
                             sem_d)
    stage_v[...] = carry
    cp_t = pltpu.async_copy(stage_v, tot_s.at[pl.ds(wid * 16, 16)], sem_d)
    cp_sf.wait()
    cp_t.wait()

    plsc.subcore_barrier()

    cp_x = pltpu.async_copy(sfx_s, sfx_v, sem_d)
    pltpu.sync_copy(tot_s, tot_v)
    tots = plsc.load_gather(tot_v, [lanes * 16])
    rt = lax.rev(tots, (0,))
    cs = plsc.cumsum(rt)
    offs_v[...] = lax.rev(cs - rt, (0,))
    cp_x.wait()
    cp_ie.wait()

    acc = zeros16
    ecnt = zeros16
    for c in range(PER_TILE // 16):
        src = idx_a if c < 8 else idx_b
        ic = src[pl.ds((c % 8) * 16, 16)]
        s = plsc.load_gather(sfx_v, [ic]) + plsc.load_gather(offs_v, [ic >> 6])
        lse = _log16(s)
        iec = ie_v[pl.ds(c * 16, 16)]
        acc = acc + iec * (lse - h_v[pl.ds(c * 16, 16)])
        ecnt = ecnt + iec
    p_vec = _lane(plsc.cumsum(acc), 15)
    e_vec = _lane(plsc.cumsum(ecnt), 15)
    stage_v[...] = jnp.where(lanes == 0, p_vec,
                             jnp.where(lanes == 1, e_vec, zeros16))
    sp = pltpu.async_copy(stage_v, pacc_s.at[idx16_v], sem_d, add=True)
    sp.wait()

    plsc.subcore_barrier()

    @pl.when(wid == 0)
    def _():
        pltpu.sync_copy(pacc_s, z_v.at[pl.ds(0, 16)])
        tot = z_v[pl.ds(0, 16)]
        out_v[...] = _lane(tot, 0) / (_lane(tot, 1) + EPSILON)
        pltpu.sync_copy(out_v, out_hbm)


def kernel(hazard, is_event, event_time):
    hazard = hazard.reshape(-1).astype(jnp.float32)
    is_event = is_event.reshape(-1).astype(jnp.float32)
    et = event_time.reshape(-1).astype(jnp.int32)

    mesh = plsc.VectorSubcoreMesh(core_axis_name="c", subcore_axis_name="s",
                                  num_cores=1)
    run = pl.kernel(
        _cox_body,
        out_type=jax.ShapeDtypeStruct((16,), jnp.float32),
        mesh=mesh,
        compiler_params=pltpu.CompilerParams(needs_layout_passes=False),
        scratch_types=[
            pltpu.VMEM((PER_TILE,), jnp.float32),
            pltpu.VMEM((PER_TILE,), jnp.float32),
            pltpu.VMEM((128,), jnp.int32),
            pltpu.VMEM((128,), jnp.int32),
            pltpu.VMEM((128,), jnp.float32),
            pltpu.VMEM((128,), jnp.float32),
            pltpu.VMEM((16,), jnp.int32),
            pltpu.VMEM((STRIPE,), jnp.float32),
            pltpu.VMEM((STRIPE,), jnp.float32),
            pltpu.VMEM((NBUCKETS,), jnp.float32),
            pltpu.VMEM((NTILES * 16,), jnp.float32),
            pltpu.VMEM((16,), jnp.float32),
            pltpu.VMEM((16,), jnp.float32),
            pltpu.VMEM((16,), jnp.float32),
            pltpu.VMEM_SHARED((NBUCKETS,), jnp.float32),
            pltpu.VMEM_SHARED((NBUCKETS,), jnp.float32),
            pltpu.VMEM_SHARED((NTILES * 16,), jnp.float32),
            pltpu.VMEM_SHARED((16,), jnp.float32),
            pltpu.SemaphoreType.DMA,
            pltpu.SemaphoreType.DMA,
            pltpu.SemaphoreType.DMA,
            pltpu.SemaphoreType.DMA,
        ],
    )
    out = run(hazard, is_event, et)
    return out


# speedup vs baseline: 1.0007x; 1.0007x over previous
import jax
import jax.numpy as jnp
from jax import lax
from jax.experimental import pallas as pl
from jax.experimental.pallas import tpu as pltpu
from jax.experimental.pallas import tpu_sc as plsc

N = 4096
NBUCKETS = 1024
NTILES = 16
PER_TILE = N // NTILES
STRIPE = NBUCKETS // NTILES
LN2 = 0.6931471805599453
EPSILON = 1e-07


def _lane(v, i):
    dn = lax.GatherDimensionNumbers(offset_dims=(), collapsed_slice_dims=(0,),
                                    start_index_map=(0,))
    idx = jnp.full((16, 1), i, jnp.int32)
    return lax.gather(v, idx, dn, slice_sizes=(1,),
                      mode=lax.GatherScatterMode.PROMISE_IN_BOUNDS)


def _log16(s):
    bits = plsc.bitcast(s, jnp.int32)
    ex = ((bits >> 23) & 0xFF) - 127
    m = plsc.bitcast((bits & 0x7FFFFF) | 0x3F800000, jnp.float32)
    big = m > 1.5
    m = jnp.where(big, m * 0.5, m)
    ex = jnp.where(big, ex + 1, ex)
    z = (m - 1.0) / (m + 1.0)
    z2 = z * z
    p = jnp.full((16,), 1.0 / 9.0, jnp.float32)
    for c in (1.0 / 7.0, 1.0 / 5.0, 1.0 / 3.0, 1.0):
        p = p * z2 + c
    return ex.astype(jnp.float32) * LN2 + 2.0 * z * p


def _cox_body(h_hbm, ie_hbm, et_hbm, out_hbm,
              h_v, ie_v, idx_a, idx_b, e_a, e_b, idx16_v,
              z_v, loc_v, loc2_v, sfx_v, tot_v, offs_v, stage_v, out_v,
              hist_s, hist2_s, sfx_s, tot_s, pacc_s,
              sem_a, sem_b, sem_c, sem_d):
    wid = lax.axis_index("s")
    base = wid * PER_TILE
    lanes = lax.iota(jnp.int32, 16)
    zeros16 = jnp.zeros((16,), jnp.float32)

    cp_h = pltpu.async_copy(h_hbm.at[pl.ds(base, PER_TILE)], h_v, sem_a)
    cp_i1 = pltpu.async_copy(et_hbm.at[pl.ds(base, 128)], idx_a, sem_b)
    cp_i2 = pltpu.async_copy(et_hbm.at[pl.ds(base + 128, 128)], idx_b, sem_b)
    cp_ie = pltpu.async_copy(ie_hbm.at[pl.ds(base, PER_TILE)], ie_v, sem_c)

    for k in range(4):
        z_v[pl.ds(k * 16, 16)] = zeros16
    idx16_v[...] = lanes
    cp_z = pltpu.async_copy(z_v, hist_s.at[pl.ds(wid * STRIPE, STRIPE)], sem_d)
    cp_z2 = pltpu.async_copy(z_v, hist2_s.at[pl.ds(wid * STRIPE, STRIPE)],
                             sem_d)

    @pl.when(wid == 0)
    def _():
        pltpu.sync_copy(z_v.at[pl.ds(0, 16)], pacc_s)

    cp_h.wait()
    for c in range(PER_TILE // 16):
        ec = jnp.exp(h_v[pl.ds(c * 16, 16)])
        if c < 8:
            e_a[pl.ds(c * 16, 16)] = ec
        else:
            e_b[pl.ds((c - 8) * 16, 16)] = ec
    cp_z.wait()
    cp_z2.wait()

    plsc.subcore_barrier()

    cp_i1.wait()
    cp_i2.wait()
    s1 = pltpu.async_copy(e_a, hist_s.at[idx_a], sem_d, add=True)
    s2 = pltpu.async_copy(e_b, hist2_s.at[idx_b], sem_d, add=True)
    s1.wait()
    s2.wait()

    plsc.subcore_barrier()

    cp_l1 = pltpu.async_copy(hist_s.at[pl.ds(wid * STRIPE, STRIPE)], loc_v,
                             sem_d)
    cp_l2 = pltpu.async_copy(hist2_s.at[pl.ds(wid * STRIPE, STRIPE)], loc2_v,
                             sem_d)
    cp_l1.wait()
    cp_l2.wait()
    for c in range(STRIPE // 16):
        loc_v[pl.ds(c * 16, 16)] = (loc_v[pl.ds(c * 16, 16)]
                                    + loc2_v[pl.ds(c * 16, 16)])
    carry = zeros16
    for c in range(STRIPE // 16 - 1, -1, -1):
        v = loc_v[pl.ds(c * 16, 16)]
        cs = plsc.cumsum(lax.rev(v, (0,)))
        loc_v[pl.ds(c * 16, 16)] = lax.rev(cs, (0,)) + carry
        carry = carry + _lane(cs, 15)
    cp_sf = pltpu.async_copy(loc_v, sfx_s.at[pl.ds(wid * STRIPE, STRIPE)],
                             sem_d)
    stage_v[...] = carry
    cp_t = pltpu.async_copy(stage_v, tot_s.at[pl.ds(wid * 16, 16)], sem_d)
    cp_sf.wait()
    cp_t.wait()

    plsc.subcore_barrier()

    cp_x = pltpu.async_copy(sfx_s, sfx_v, sem_d)
    pltpu.sync_copy(tot_s, tot_v)
    tots = plsc.load_gather(tot_v, [lanes * 16])
    rt = lax.rev(tots, (0,))
    cs = plsc.cumsum(rt)
    offs_v[...] = lax.rev(cs - rt, (0,))
    cp_x.wait()
    cp_ie.wait()

    acc = zeros16
    ecnt = zeros16
    for c in range(PER_TILE // 16):
        src = idx_a if c < 8 else idx_b
        ic = src[pl.ds((c % 8) * 16, 16)]
        s = plsc.load_gather(sfx_v, [ic]) + plsc.load_gather(offs_v, [ic >> 6])
        lse = _log16(s)
        iec = ie_v[pl.ds(c * 16, 16)]
        acc = acc + iec * (lse - h_v[pl.ds(c * 16, 16)])
        ecnt = ecnt + iec
    p_vec = _lane(plsc.cumsum(acc), 15)
    e_vec = _lane(plsc.cumsum(ecnt), 15)
    stage_v[...] = jnp.where(lanes == 0, p_vec,
                             jnp.where(lanes == 1, e_vec, zeros16))
    sp = pltpu.async_copy(stage_v, pacc_s.at[idx16_v], sem_d, add=True)
    sp.wait()

    plsc.subcore_barrier()

    @pl.when(wid == 0)
    def _():
        pltpu.sync_copy(pacc_s, z_v.at[pl.ds(0, 16)])
        tot = z_v[pl.ds(0, 16)]
        out_v[...] = _lane(tot, 0) / (_lane(tot, 1) + EPSILON)
        pltpu.sync_copy(out_v, out_hbm)


def kernel(hazard, is_event, event_time):
    hazard = hazard.reshape(-1).astype(jnp.float32)
    is_event = is_event.reshape(-1).astype(jnp.float32)
    et = event_time.reshape(-1).astype(jnp.int32)

    mesh = plsc.VectorSubcoreMesh(core_axis_name="c", subcore_axis_name="s",
                                  num_cores=1)
    run = pl.kernel(
        _cox_body,
        out_type=jax.ShapeDtypeStruct((16,), jnp.float32),
        mesh=mesh,
        compiler_params=pltpu.CompilerParams(needs_layout_passes=False),
        scratch_types=[
            pltpu.VMEM((PER_TILE,), jnp.float32),
            pltpu.VMEM((PER_TILE,), jnp.float32),
            pltpu.VMEM((128,), jnp.int32),
            pltpu.VMEM((128,), jnp.int32),
            pltpu.VMEM((128,), jnp.float32),
            pltpu.VMEM((128,), jnp.float32),
            pltpu.VMEM((16,), jnp.int32),
            pltpu.VMEM((STRIPE,), jnp.float32),
            pltpu.VMEM((STRIPE,), jnp.float32),
            pltpu.VMEM((STRIPE,), jnp.float32),
            pltpu.VMEM((NBUCKETS,), jnp.float32),
            pltpu.VMEM((NTILES * 16,), jnp.float32),
            pltpu.VMEM((16,), jnp.float32),
            pltpu.VMEM((16,), jnp.float32),
            pltpu.VMEM((16,), jnp.float32),
            pltpu.VMEM_SHARED((NBUCKETS,), jnp.float32),
            pltpu.VMEM_SHARED((NBUCKETS,), jnp.float32),
            pltpu.VMEM_SHARED((NBUCKETS,), jnp.float32),
            pltpu.VMEM_SHARED((NTILES * 16,), jnp.float32),
            pltpu.VMEM_SHARED((16,), jnp.float32),
            pltpu.SemaphoreType.DMA,
            pltpu.SemaphoreType.DMA,
            pltpu.SemaphoreType.DMA,
            pltpu.SemaphoreType.DMA,
        ],
    )
    out = run(hazard, is_event, et)
    return out[0]


# speedup vs baseline: 1.0035x; 1.0028x over previous
import jax
import jax.numpy as jnp
from jax import lax
from jax.experimental import pallas as pl
from jax.experimental.pallas import tpu as pltpu
from jax.experimental.pallas import tpu_sc as plsc

N = 4096
NBUCKETS = 1024
NTILES = 16
PER_TILE = N // NTILES
STRIPE = NBUCKETS // NTILES
LN2 = 0.6931471805599453
EPSILON = 1e-07


def _lane(v, i):
    dn = lax.GatherDimensionNumbers(offset_dims=(), collapsed_slice_dims=(0,),
                                    start_index_map=(0,))
    idx = jnp.full((16, 1), i, jnp.int32)
    return lax.gather(v, idx, dn, slice_sizes=(1,),
                      mode=lax.GatherScatterMode.PROMISE_IN_BOUNDS)


def _log16(s):
    bits = plsc.bitcast(s, jnp.int32)
    ex = ((bits >> 23) & 0xFF) - 127
    m = plsc.bitcast((bits & 0x7FFFFF) | 0x3F800000, jnp.float32)
    big = m > 1.5
    m = jnp.where(big, m * 0.5, m)
    ex = jnp.where(big, ex + 1, ex)
    z = (m - 1.0) / (m + 1.0)
    z2 = z * z
    p = jnp.full((16,), 1.0 / 7.0, jnp.float32)
    for c in (1.0 / 5.0, 1.0 / 3.0, 1.0):
        p = p * z2 + c
    return ex.astype(jnp.float32) * LN2 + 2.0 * z * p


def _cox_body(h_hbm, ie_hbm, et_hbm, out_hbm,
              h_v, ie_v, idx_a, idx_b, e_a, e_b, idx16_v,
              z_v, loc_v, loc2_v, sfx_v, tot_v, offs_v, stage_v, out_v,
              hist_s, hist2_s, sfx_s, tot_s, pacc_s,
              sem_a, sem_b, sem_c, sem_d):
    wid = lax.axis_index("s")
    base = wid * PER_TILE
    lanes = lax.iota(jnp.int32, 16)
    zeros16 = jnp.zeros((16,), jnp.float32)

    cp_h = pltpu.async_copy(h_hbm.at[pl.ds(base, PER_TILE)], h_v, sem_a)
    cp_i1 = pltpu.async_copy(et_hbm.at[pl.ds(base, 128)], idx_a, sem_b)
    cp_i2 = pltpu.async_copy(et_hbm.at[pl.ds(base + 128, 128)], idx_b, sem_b)
    cp_ie = pltpu.async_copy(ie_hbm.at[pl.ds(base, PER_TILE)], ie_v, sem_c)

    for k in range(4):
        z_v[pl.ds(k * 16, 16)] = zeros16
    idx16_v[...] = lanes
    cp_z = pltpu.async_copy(z_v, hist_s.at[pl.ds(wid * STRIPE, STRIPE)], sem_d)
    cp_z2 = pltpu.async_copy(z_v, hist2_s.at[pl.ds(wid * STRIPE, STRIPE)],
                             sem_d)

    @pl.when(wid == 0)
    def _():
        pltpu.sync_copy(z_v.at[pl.ds(0, 16)], pacc_s)

    cp_z.wait()
    cp_z2.wait()

    plsc.subcore_barrier()

    cp_h.wait()
    for c in range(PER_TILE // 16):
        ec = jnp.exp(h_v[pl.ds(c * 16, 16)])
        if c < 8:
            e_a[pl.ds(c * 16, 16)] = ec
        else:
            e_b[pl.ds((c - 8) * 16, 16)] = ec

    cp_i1.wait()
    cp_i2.wait()
    s1 = pltpu.async_copy(e_a, hist_s.at[idx_a], sem_d, add=True)
    s2 = pltpu.async_copy(e_b, hist2_s.at[idx_b], sem_d, add=True)
    s1.wait()
    s2.wait()

    plsc.subcore_barrier()

    cp_l1 = pltpu.async_copy(hist_s.at[pl.ds(wid * STRIPE, STRIPE)], loc_v,
                             sem_d)
    cp_l2 = pltpu.async_copy(hist2_s.at[pl.ds(wid * STRIPE, STRIPE)], loc2_v,
                             sem_d)
    cp_l1.wait()
    cp_l2.wait()
    for c in range(STRIPE // 16):
        loc_v[pl.ds(c * 16, 16)] = (loc_v[pl.ds(c * 16, 16)]
                                    + loc2_v[pl.ds(c * 16, 16)])
    carry = zeros16
    for c in range(STRIPE // 16 - 1, -1, -1):
        v = loc_v[pl.ds(c * 16, 16)]
        cs = plsc.cumsum(lax.rev(v, (0,)))
        loc_v[pl.ds(c * 16, 16)] = lax.rev(cs, (0,)) + carry
        carry = carry + _lane(cs, 15)
    cp_sf = pltpu.async_copy(loc_v, sfx_s.at[pl.ds(wid * STRIPE, STRIPE)],
                             sem_d)
    stage_v[...] = carry
    cp_t = pltpu.async_copy(stage_v, tot_s.at[pl.ds(wid * 16, 16)], sem_d)
    cp_sf.wait()
    cp_t.wait()

    plsc.subcore_barrier()

    cp_x = pltpu.async_copy(sfx_s, sfx_v, sem_d)
    cp_tt = pltpu.async_copy(tot_s, tot_v, sem_a)
    cp_tt.wait()
    tots = plsc.load_gather(tot_v, [lanes * 16])
    rt = lax.rev(tots, (0,))
    cs = plsc.cumsum(rt)
    offs_v[...] = lax.rev(cs - rt, (0,))
    cp_x.wait()
    cp_ie.wait()

    acc = zeros16
    ecnt = zeros16
    for c in range(PER_TILE // 16):
        src = idx_a if c < 8 else idx_b
        ic = src[pl.ds((c % 8) * 16, 16)]
        s = plsc.load_gather(sfx_v, [ic]) + plsc.load_gather(offs_v, [ic >> 6])
        lse = _log16(s)
        iec = ie_v[pl.ds(c * 16, 16)]
        acc = acc + iec * (lse - h_v[pl.ds(c * 16, 16)])
        ecnt = ecnt + iec
    p_vec = _lane(plsc.cumsum(acc), 15)
    e_vec = _lane(plsc.cumsum(ecnt), 15)
    stage_v[...] = jnp.where(lanes == 0, p_vec,
                             jnp.where(lanes == 1, e_vec, zeros16))
    sp = pltpu.async_copy(stage_v, pacc_s.at[idx16_v], sem_d, add=True)
    sp.wait()

    plsc.subcore_barrier()

    @pl.when(wid == 0)
    def _():
        pltpu.sync_copy(pacc_s, z_v.at[pl.ds(0, 16)])
        tot = z_v[pl.ds(0, 16)]
        out_v[...] = _lane(tot, 0) / (_lane(tot, 1) + EPSILON)
        pltpu.sync_copy(out_v, out_hbm)


def kernel(hazard, is_event, event_time):
    hazard = hazard.reshape(-1).astype(jnp.float32)
    is_event = is_event.reshape(-1).astype(jnp.float32)
    et = event_time.reshape(-1).astype(jnp.int32)

    mesh = plsc.VectorSubcoreMesh(core_axis_name="c", subcore_axis_name="s",
                                  num_cores=1)
    run = pl.kernel(
        _cox_body,
        out_type=jax.ShapeDtypeStruct((16,), jnp.float32),
        mesh=mesh,
        compiler_params=pltpu.CompilerParams(needs_layout_passes=False),
        scratch_types=[
            pltpu.VMEM((PER_TILE,), jnp.float32),
            pltpu.VMEM((PER_TILE,), jnp.float32),
            pltpu.VMEM((128,), jnp.int32),
            pltpu.VMEM((128,), jnp.int32),
            pltpu.VMEM((128,), jnp.float32),
            pltpu.VMEM((128,), jnp.float32),
            pltpu.VMEM((16,), jnp.int32),
            pltpu.VMEM((STRIPE,), jnp.float32),
            pltpu.VMEM((STRIPE,), jnp.float32),
            pltpu.VMEM((STRIPE,), jnp.float32),
            pltpu.VMEM((NBUCKETS,), jnp.float32),
            pltpu.VMEM((NTILES * 16,), jnp.float32),
            pltpu.VMEM((16,), jnp.float32),
            pltpu.VMEM((16,), jnp.float32),
            pltpu.VMEM((16,), jnp.float32),
            pltpu.VMEM_SHARED((NBUCKETS,), jnp.float32),
            pltpu.VMEM_SHARED((NBUCKETS,), jnp.float32),
            pltpu.VMEM_SHARED((NBUCKETS,), jnp.float32),
            pltpu.VMEM_SHARED((NTILES * 16,), jnp.float32),
            pltpu.VMEM_SHARED((16,), jnp.float32),
            pltpu.SemaphoreType.DMA,
            pltpu.SemaphoreType.DMA,
            pltpu.SemaphoreType.DMA,
            pltpu.SemaphoreType.DMA,
        ],
    )
    out = run(hazard, is_event, et)
    return out[0]
